# baseline (device time: 23477 ns/iter reference)
import jax
import jax.numpy as jnp
from jax import lax
from jax.experimental import pallas as pl
from jax.experimental.pallas import tpu as pltpu

N_DEV = 8
N_HALF = 2


def _gelu(z):
    return 0.5 * z * (1.0 + jnp.tanh(0.7978845608 * (z + 0.044715 * z * z * z)))


def kernel(A, B):
    m, k = A.shape
    k2, n = B.shape
    blk = m // N_DEV
    hn = n // N_HALF

    def body(a_ref, b_ref, out_ref, rs_recv0, rs_recv1,
             rs_send_sems, rs_recv_sems, ag_send_sems, ag_recv_sems):
        my_pos = lax.axis_index("i")
        rs_recv = [rs_recv0, rs_recv1]

        barrier_sem = pltpu.get_barrier_semaphore()
        for j in range(1, N_DEV):
            pl.semaphore_signal(
                barrier_sem, inc=1,
                device_id=((my_pos + j) % N_DEV,),
                device_id_type=pl.DeviceIdType.MESH,
            )

        rs_rdmas = [[], []]
        for j in range(N_DEV - 1):
            peer = (my_pos + 1 + j) % N_DEV
            p_off = pl.multiple_of(peer * blk, blk)
            out_ref[pl.ds(p_off, blk), :] = jnp.dot(
                a_ref[pl.ds(p_off, blk), :], b_ref[:, :],
                preferred_element_type=jnp.float32)
            if j == 0:
                pl.semaphore_wait(barrier_sem, N_DEV - 1)
            for h in range(N_HALF):
                rdma = pltpu.make_async_remote_copy(
                    src_ref=out_ref.at[pl.ds(p_off, blk), pl.ds(h * hn, hn)],
                    dst_ref=rs_recv[h].at[N_DEV - 2 - j],
                    send_sem=rs_send_sems.at[h * (N_DEV - 1) + j],
                    recv_sem=rs_recv_sems.at[h * (N_DEV - 1) + (N_DEV - 2 - j)],
                    device_id=(peer,),
                    device_id_type=pl.DeviceIdType.MESH,
                )
                rdma.start()
                rs_rdmas[h].append(rdma)

        my_off = pl.multiple_of(my_pos * blk, blk)
        out_ref[pl.ds(my_off, blk), :] = jnp.dot(
            a_ref[pl.ds(my_off, blk), :], b_ref[:, :],
            preferred_element_type=jnp.float32)

        ag_rdmas = []
        for h in range(N_HALF):
            for rdma in rs_rdmas[h]:
                rdma.wait_recv()
            cols = pl.ds(h * hn, hn)
            block = out_ref[pl.ds(my_off, blk), cols]
            for s in range(N_DEV - 1):
                block += rs_recv[h][s, :, :]
            out_ref[pl.ds(my_off, blk), cols] = _gelu(block)
            for j in range(N_DEV - 1):
                peer = (my_pos + 1 + j) % N_DEV
                rdma = pltpu.make_async_remote_copy(
                    src_ref=out_ref.at[pl.ds(my_off, blk), cols],
                    dst_ref=out_ref.at[pl.ds(my_off, blk), cols],
                    send_sem=ag_send_sems.at[h * (N_DEV - 1) + j],
                    recv_sem=ag_recv_sems.at[h * (N_DEV - 1) + (N_DEV - 2 - j)],
                    device_id=(peer,),
                    device_id_type=pl.DeviceIdType.MESH,
                )
                rdma.start()
                ag_rdmas.append(rdma)

        for rdma in ag_rdmas:
            rdma.wait_recv()
        for rdma in rs_rdmas[0] + rs_rdmas[1] + ag_rdmas:
            rdma.wait_send()

    n_sem = N_HALF * (N_DEV - 1)
    return pl.pallas_call(
        body,
        out_shape=jax.ShapeDtypeStruct((m, n), jnp.float32),
        in_specs=[
            pl.BlockSpec(memory_space=pltpu.VMEM),
            pl.BlockSpec(memory_space=pltpu.VMEM),
        ],
        out_specs=pl.BlockSpec(memory_space=pltpu.VMEM),
        scratch_shapes=[
            pltpu.VMEM((N_DEV - 1, blk, hn), jnp.float32),
            pltpu.VMEM((N_DEV - 1, blk, hn), jnp.float32),
            pltpu.SemaphoreType.DMA((n_sem,)),
            pltpu.SemaphoreType.DMA((n_sem,)),
            pltpu.SemaphoreType.DMA((n_sem,)),
            pltpu.SemaphoreType.DMA((n_sem,)),
        ],
        compiler_params=pltpu.CompilerParams(collective_id=0),
    )(A, B)
